# baseline (speedup 1.0000x reference)
import functools

import jax
import jax.numpy as jnp
from jax.experimental import pallas as pl
from jax.experimental.pallas import tpu as pltpu

D_MODEL = 1024
N_CLUSTERS = 64
TOP_K = 8
BASE_TEMP = 2.0
SEQ_LEN = 2048
PRED_LEN = 512

_TEMP = BASE_TEMP * (1.0 + PRED_LEN / SEQ_LEN)
_INV_TEMP = 1.0 / _TEMP


def _fused_kernel(x_ref, d_ref, xc_ref, xr_ref, aux_ref, q_buf, acc_ref,
                  w_buf, *, tile_rows, half_rows, n_rows, n_tiles):
    i = pl.program_id(0)
    j = pl.program_id(1)
    d = d_ref[...]

    @pl.when(j == 0)
    def _():
        x_t = x_ref[...]
        scores_t = jax.lax.dot_general(
            d, x_t, (((1,), (1,)), ((), ())),
            preferred_element_type=jnp.float32)
        st = scores_t * _INV_TEMP

        m0 = jnp.max(st, axis=0, keepdims=True)
        e = jnp.exp(st - m0)
        q = e * (1.0 / jnp.sum(e, axis=0, keepdims=True))
        q_buf[:, pl.ds(i * tile_rows, tile_rows)] = q

        @pl.when(i == 0)
        def _():
            acc_ref[...] = q

        @pl.when(i > 0)
        def _():
            acc_ref[...] = acc_ref[...] + q

        k = st.shape[0]
        iota = jax.lax.broadcasted_iota(jnp.int32, st.shape, 0)
        work = st
        neg_inf = jnp.float32(-jnp.inf)
        m = m0
        for _r in range(TOP_K):
            is_m = work == m
            idx = jnp.min(jnp.where(is_m, iota, k), axis=0, keepdims=True)
            work = jnp.where(iota == idx, neg_inf, work)
            if _r < TOP_K - 1:
                m = jnp.max(work, axis=0, keepdims=True)

        ew = jnp.where(work == neg_inf, e, 0.0)
        w = ew * (1.0 / jnp.sum(ew, axis=0, keepdims=True))
        w_buf[0] = w[:, :half_rows]
        w_buf[1] = w[:, half_rows:]

    w_j = w_buf[j]
    xc = jax.lax.dot_general(
        w_j, d, (((0,), (0,)), ((), ())),
        preferred_element_type=jnp.float32)
    xc_ref[...] = xc
    xr_ref[...] = x_ref[pl.ds(j * half_rows, half_rows), :] - xc

    @pl.when(jnp.logical_and(i == n_tiles - 1, j == 1))
    def _():
        qf = q_buf[...]
        csum = jnp.sum(acc_ref[...], axis=1, keepdims=True)
        weight = (qf * qf) / csum
        rowsum = jnp.sum(weight, axis=0, keepdims=True)
        p = weight / rowsum
        kl_elem = p * (jnp.log(qf) - jnp.log(csum) - jnp.log(rowsum))
        kl = jnp.sum(kl_elem) / n_rows

        gram = jax.lax.dot_general(
            d, d, (((1,), (1,)), ((), ())),
            preferred_element_type=jnp.float32)
        kk = gram.shape[0]
        r_i = jax.lax.broadcasted_iota(jnp.int32, gram.shape, 0)
        c_i = jax.lax.broadcasted_iota(jnp.int32, gram.shape, 1)
        eye = jnp.where(r_i == c_i, 1.0, 0.0).astype(gram.dtype)
        diff = gram - eye
        ortho = jnp.sum(diff * diff) / (kk * kk)

        aux_ref[0, 0] = kl * (SEQ_LEN / PRED_LEN) + 0.1 * ortho


def kernel(x, dictionary):
    B, N, D = x.shape
    K = dictionary.shape[0]
    n_rows = B * N
    tile_rows = 1024
    half_rows = tile_rows // 2
    n_tiles = n_rows // tile_rows
    xf = x.reshape(n_rows, D)

    out_types = (
        jax.ShapeDtypeStruct((n_rows, D), jnp.float32),
        jax.ShapeDtypeStruct((n_rows, D), jnp.float32),
        jax.ShapeDtypeStruct((1, 1), jnp.float32),
    )
    xc, xr, aux = pl.pallas_call(
        functools.partial(_fused_kernel, tile_rows=tile_rows,
                          half_rows=half_rows, n_rows=n_rows,
                          n_tiles=n_tiles),
        grid=(n_tiles, 2),
        in_specs=[
            pl.BlockSpec((tile_rows, D), lambda i, j: (i, 0)),
            pl.BlockSpec((K, D), lambda i, j: (0, 0)),
        ],
        out_specs=(
            pl.BlockSpec((half_rows, D), lambda i, j: (2 * i + j, 0)),
            pl.BlockSpec((half_rows, D), lambda i, j: (2 * i + j, 0)),
            pl.BlockSpec(memory_space=pltpu.SMEM),
        ),
        out_shape=out_types,
        scratch_shapes=[
            pltpu.VMEM((K, n_rows), jnp.float32),
            pltpu.VMEM((K, tile_rows), jnp.float32),
            pltpu.VMEM((2, K, half_rows), jnp.float32),
        ],
    )(xf, dictionary)

    return (xc.reshape(B, N, D), xr.reshape(B, N, D), aux[0, 0])


# speedup vs baseline: 1.0882x; 1.0882x over previous
import functools

import jax
import jax.numpy as jnp
from jax.experimental import pallas as pl
from jax.experimental.pallas import tpu as pltpu

D_MODEL = 1024
N_CLUSTERS = 64
TOP_K = 8
BASE_TEMP = 2.0
SEQ_LEN = 2048
PRED_LEN = 512

_TEMP = BASE_TEMP * (1.0 + PRED_LEN / SEQ_LEN)
_INV_TEMP = 1.0 / _TEMP


def _fused_kernel(x_ref, d_ref, xc_ref, xr_ref, aux_ref, q_buf, acc_ref,
                  *, step_rows, n_rows, n_steps):
    s = pl.program_id(0)
    d = d_ref[...]
    x_t = x_ref[pl.ds((s % 2) * step_rows, step_rows), :]

    scores_t = jax.lax.dot_general(
        d, x_t, (((1,), (1,)), ((), ())),
        preferred_element_type=jnp.float32)
    st = scores_t * _INV_TEMP

    m0 = jnp.max(st, axis=0, keepdims=True)
    e = jnp.exp(st - m0)
    q = e * (1.0 / jnp.sum(e, axis=0, keepdims=True))
    q_buf[:, pl.ds(s * step_rows, step_rows)] = q

    @pl.when(s == 0)
    def _():
        acc_ref[...] = q

    @pl.when(s > 0)
    def _():
        acc_ref[...] = acc_ref[...] + q

    k = st.shape[0]
    iota = jax.lax.broadcasted_iota(jnp.int32, st.shape, 0)
    work = st
    neg_inf = jnp.float32(-jnp.inf)
    m = m0
    for _r in range(TOP_K):
        is_m = work == m
        idx = jnp.min(jnp.where(is_m, iota, k), axis=0, keepdims=True)
        work = jnp.where(iota == idx, neg_inf, work)
        if _r < TOP_K - 1:
            m = jnp.max(work, axis=0, keepdims=True)

    ew = jnp.where(work == neg_inf, e, 0.0)
    w = ew * (1.0 / jnp.sum(ew, axis=0, keepdims=True))

    xc = jax.lax.dot_general(
        w, d, (((0,), (0,)), ((), ())),
        preferred_element_type=jnp.float32)
    xc_ref[...] = xc
    xr_ref[...] = x_t - xc

    @pl.when(s == n_steps - 1)
    def _():
        qf = q_buf[...]
        csum = jnp.sum(acc_ref[...], axis=1, keepdims=True)
        weight = (qf * qf) / csum
        rowsum = jnp.sum(weight, axis=0, keepdims=True)
        p = weight / rowsum
        kl_elem = p * (jnp.log(qf) - jnp.log(csum) - jnp.log(rowsum))
        kl = jnp.sum(kl_elem) / n_rows

        gram = jax.lax.dot_general(
            d, d, (((1,), (1,)), ((), ())),
            preferred_element_type=jnp.float32)
        kk = gram.shape[0]
        r_i = jax.lax.broadcasted_iota(jnp.int32, gram.shape, 0)
        c_i = jax.lax.broadcasted_iota(jnp.int32, gram.shape, 1)
        eye = jnp.where(r_i == c_i, 1.0, 0.0).astype(gram.dtype)
        diff = gram - eye
        ortho = jnp.sum(diff * diff) / (kk * kk)

        aux_ref[0, 0] = kl * (SEQ_LEN / PRED_LEN) + 0.1 * ortho


def kernel(x, dictionary):
    B, N, D = x.shape
    K = dictionary.shape[0]
    n_rows = B * N
    step_rows = 512
    in_rows = 2 * step_rows
    n_steps = n_rows // step_rows
    xf = x.reshape(n_rows, D)

    out_types = (
        jax.ShapeDtypeStruct((n_rows, D), jnp.float32),
        jax.ShapeDtypeStruct((n_rows, D), jnp.float32),
        jax.ShapeDtypeStruct((1, 1), jnp.float32),
    )
    xc, xr, aux = pl.pallas_call(
        functools.partial(_fused_kernel, step_rows=step_rows,
                          n_rows=n_rows, n_steps=n_steps),
        grid=(n_steps,),
        in_specs=[
            pl.BlockSpec((in_rows, D), lambda s: (s // 2, 0)),
            pl.BlockSpec((K, D), lambda s: (0, 0)),
        ],
        out_specs=(
            pl.BlockSpec((step_rows, D), lambda s: (s, 0)),
            pl.BlockSpec((step_rows, D), lambda s: (s, 0)),
            pl.BlockSpec(memory_space=pltpu.SMEM),
        ),
        out_shape=out_types,
        scratch_shapes=[
            pltpu.VMEM((K, n_rows), jnp.float32),
            pltpu.VMEM((K, step_rows), jnp.float32),
        ],
    )(xf, dictionary)

    return (xc.reshape(B, N, D), xr.reshape(B, N, D), aux[0, 0])


# speedup vs baseline: 1.3813x; 1.2693x over previous
import functools

import jax
import jax.numpy as jnp
from jax.experimental import pallas as pl
from jax.experimental.pallas import tpu as pltpu

D_MODEL = 1024
N_CLUSTERS = 64
TOP_K = 8
BASE_TEMP = 2.0
SEQ_LEN = 2048
PRED_LEN = 512

_TEMP = BASE_TEMP * (1.0 + PRED_LEN / SEQ_LEN)
_INV_TEMP = 1.0 / _TEMP


def _fused_kernel(x_ref, d_ref, xc_ref, xr_ref, aux_ref, q_buf, acc_ref,
                  *, tile_rows, n_rows, n_tiles):
    i = pl.program_id(0)
    x_t = x_ref[...]
    d = d_ref[...]

    scores_t = jax.lax.dot_general(
        d, x_t, (((1,), (1,)), ((), ())),
        preferred_element_type=jnp.float32)
    st = scores_t * _INV_TEMP

    m0 = jnp.max(st, axis=0, keepdims=True)
    e = jnp.exp(st - m0)
    q = e * (1.0 / jnp.sum(e, axis=0, keepdims=True))
    q_buf[:, pl.ds(i * tile_rows, tile_rows)] = q

    @pl.when(i == 0)
    def _():
        acc_ref[...] = q

    @pl.when(i > 0)
    def _():
        acc_ref[...] = acc_ref[...] + q

    k = st.shape[0]
    iota = jax.lax.broadcasted_iota(jnp.int32, st.shape, 0)
    work = st
    neg_inf = jnp.float32(-jnp.inf)
    m = m0
    for _r in range(TOP_K):
        is_m = work == m
        idx = jnp.min(jnp.where(is_m, iota, k), axis=0, keepdims=True)
        work = jnp.where(iota == idx, neg_inf, work)
        if _r < TOP_K - 1:
            m = jnp.max(work, axis=0, keepdims=True)

    ew = jnp.where(work == neg_inf, e, 0.0)
    w = ew * (1.0 / jnp.sum(ew, axis=0, keepdims=True))

    xc = jax.lax.dot_general(
        w, d, (((0,), (0,)), ((), ())),
        preferred_element_type=jnp.float32)
    xc_ref[...] = xc
    xr_ref[...] = x_t - xc

    @pl.when(i == 1)
    def _():
        gram = jax.lax.dot_general(
            d, d, (((1,), (1,)), ((), ())),
            preferred_element_type=jnp.float32)
        kk = gram.shape[0]
        r_i = jax.lax.broadcasted_iota(jnp.int32, gram.shape, 0)
        c_i = jax.lax.broadcasted_iota(jnp.int32, gram.shape, 1)
        eye = jnp.where(r_i == c_i, 1.0, 0.0).astype(gram.dtype)
        diff = gram - eye
        ortho = jnp.sum(diff * diff) / (kk * kk)
        aux_ref[0, 0] = 0.1 * ortho

    @pl.when(i == n_tiles - 1)
    def _():
        qf = q_buf[...]
        csum = jnp.sum(acc_ref[...], axis=1, keepdims=True)
        weight = (qf * qf) / csum
        rowsum = jnp.sum(weight, axis=0, keepdims=True)
        p = weight / rowsum
        kl_elem = p * (jnp.log(qf) - jnp.log(csum) - jnp.log(rowsum))
        kl = jnp.sum(kl_elem) / n_rows

        aux_ref[0, 0] = aux_ref[0, 0] + kl * (SEQ_LEN / PRED_LEN)


def kernel(x, dictionary):
    B, N, D = x.shape
    K = dictionary.shape[0]
    n_rows = B * N
    tile_rows = 1024
    n_tiles = n_rows // tile_rows
    xf = x.reshape(n_rows, D)

    out_types = (
        jax.ShapeDtypeStruct((n_rows, D), jnp.float32),
        jax.ShapeDtypeStruct((n_rows, D), jnp.float32),
        jax.ShapeDtypeStruct((1, 1), jnp.float32),
    )
    xc, xr, aux = pl.pallas_call(
        functools.partial(_fused_kernel, tile_rows=tile_rows,
                          n_rows=n_rows, n_tiles=n_tiles),
        grid=(n_tiles,),
        in_specs=[
            pl.BlockSpec((tile_rows, D), lambda i: (i, 0)),
            pl.BlockSpec((K, D), lambda i: (0, 0)),
        ],
        out_specs=(
            pl.BlockSpec((tile_rows, D), lambda i: (i, 0)),
            pl.BlockSpec((tile_rows, D), lambda i: (i, 0)),
            pl.BlockSpec(memory_space=pltpu.SMEM),
        ),
        out_shape=out_types,
        scratch_shapes=[
            pltpu.VMEM((K, n_rows), jnp.float32),
            pltpu.VMEM((K, tile_rows), jnp.float32),
        ],
    )(xf, dictionary)

    return (xc.reshape(B, N, D), xr.reshape(B, N, D), aux[0, 0])


# speedup vs baseline: 1.5074x; 1.0913x over previous
import functools

import jax
import jax.numpy as jnp
from jax.experimental import pallas as pl
from jax.experimental.pallas import tpu as pltpu

D_MODEL = 1024
N_CLUSTERS = 64
TOP_K = 8
BASE_TEMP = 2.0
SEQ_LEN = 2048
PRED_LEN = 512

_TEMP = BASE_TEMP * (1.0 + PRED_LEN / SEQ_LEN)
_INV_TEMP = 1.0 / _TEMP


def _fused_kernel(x_ref, d_ref, xc_hbm, xr_hbm, aux_ref, q_buf, acc_ref,
                  cbuf, rbuf, c_sem, r_sem,
                  *, tile_rows, n_rows, n_tiles):
    i = pl.program_id(0)
    slot = jax.lax.rem(i, 2)
    x_t = x_ref[...]
    d = d_ref[...]

    scores_t = jax.lax.dot_general(
        d, x_t, (((1,), (1,)), ((), ())),
        preferred_element_type=jnp.float32)
    st = scores_t * _INV_TEMP

    m0 = jnp.max(st, axis=0, keepdims=True)
    e = jnp.exp(st - m0)
    q = e * (1.0 / jnp.sum(e, axis=0, keepdims=True))
    q_buf[:, pl.ds(i * tile_rows, tile_rows)] = q

    @pl.when(i == 0)
    def _():
        acc_ref[...] = q

    @pl.when(i > 0)
    def _():
        acc_ref[...] = acc_ref[...] + q

    k = st.shape[0]
    iota = jax.lax.broadcasted_iota(jnp.int32, st.shape, 0)
    work = st
    neg_inf = jnp.float32(-jnp.inf)
    m = m0
    for _r in range(TOP_K):
        is_m = work == m
        idx = jnp.min(jnp.where(is_m, iota, k), axis=0, keepdims=True)
        work = jnp.where(iota == idx, neg_inf, work)
        if _r < TOP_K - 1:
            m = jnp.max(work, axis=0, keepdims=True)

    ew = jnp.where(work == neg_inf, e, 0.0)
    w = ew * (1.0 / jnp.sum(ew, axis=0, keepdims=True))

    @pl.when(i >= 2)
    def _():
        prev = i - 2
        pltpu.make_async_copy(
            cbuf.at[slot],
            xc_hbm.at[pl.ds(prev * tile_rows, tile_rows), :],
            c_sem.at[slot]).wait()
        pltpu.make_async_copy(
            rbuf.at[slot],
            xr_hbm.at[pl.ds(prev * tile_rows, tile_rows), :],
            r_sem.at[slot]).wait()

    xc = jax.lax.dot_general(
        w, d, (((0,), (0,)), ((), ())),
        preferred_element_type=jnp.float32)
    cbuf[slot] = xc
    pltpu.make_async_copy(
        cbuf.at[slot],
        xc_hbm.at[pl.ds(i * tile_rows, tile_rows), :],
        c_sem.at[slot]).start()

    rbuf[slot] = x_t - xc
    pltpu.make_async_copy(
        rbuf.at[slot],
        xr_hbm.at[pl.ds(i * tile_rows, tile_rows), :],
        r_sem.at[slot]).start()

    @pl.when(i == 1)
    def _():
        gram = jax.lax.dot_general(
            d, d, (((1,), (1,)), ((), ())),
            preferred_element_type=jnp.float32)
        kk = gram.shape[0]
        r_i = jax.lax.broadcasted_iota(jnp.int32, gram.shape, 0)
        c_i = jax.lax.broadcasted_iota(jnp.int32, gram.shape, 1)
        eye = jnp.where(r_i == c_i, 1.0, 0.0).astype(gram.dtype)
        diff = gram - eye
        ortho = jnp.sum(diff * diff) / (kk * kk)
        aux_ref[0, 0] = 0.1 * ortho

    @pl.when(i == n_tiles - 1)
    def _():
        qf = q_buf[...]
        csum = jnp.sum(acc_ref[...], axis=1, keepdims=True)
        weight = (qf * qf) / csum
        rowsum = jnp.sum(weight, axis=0, keepdims=True)
        p = weight / rowsum
        kl_elem = p * (jnp.log(qf) - jnp.log(csum) - jnp.log(rowsum))
        kl = jnp.sum(kl_elem) / n_rows
        aux_ref[0, 0] = aux_ref[0, 0] + kl * (SEQ_LEN / PRED_LEN)

        other = jax.lax.rem(i + 1, 2)
        prev = i - 1
        pltpu.make_async_copy(
            cbuf.at[other],
            xc_hbm.at[pl.ds(prev * tile_rows, tile_rows), :],
            c_sem.at[other]).wait()
        pltpu.make_async_copy(
            rbuf.at[other],
            xr_hbm.at[pl.ds(prev * tile_rows, tile_rows), :],
            r_sem.at[other]).wait()
        pltpu.make_async_copy(
            cbuf.at[slot],
            xc_hbm.at[pl.ds(i * tile_rows, tile_rows), :],
            c_sem.at[slot]).wait()
        pltpu.make_async_copy(
            rbuf.at[slot],
            xr_hbm.at[pl.ds(i * tile_rows, tile_rows), :],
            r_sem.at[slot]).wait()


def kernel(x, dictionary):
    B, N, D = x.shape
    K = dictionary.shape[0]
    n_rows = B * N
    tile_rows = 1024
    n_tiles = n_rows // tile_rows
    xf = x.reshape(n_rows, D)

    out_types = (
        jax.ShapeDtypeStruct((n_rows, D), jnp.float32),
        jax.ShapeDtypeStruct((n_rows, D), jnp.float32),
        jax.ShapeDtypeStruct((1, 1), jnp.float32),
    )
    xc, xr, aux = pl.pallas_call(
        functools.partial(_fused_kernel, tile_rows=tile_rows,
                          n_rows=n_rows, n_tiles=n_tiles),
        grid=(n_tiles,),
        in_specs=[
            pl.BlockSpec((tile_rows, D), lambda i: (i, 0)),
            pl.BlockSpec((K, D), lambda i: (0, 0)),
        ],
        out_specs=(
            pl.BlockSpec(memory_space=pl.ANY),
            pl.BlockSpec(memory_space=pl.ANY),
            pl.BlockSpec(memory_space=pltpu.SMEM),
        ),
        out_shape=out_types,
        scratch_shapes=[
            pltpu.VMEM((K, n_rows), jnp.float32),
            pltpu.VMEM((K, tile_rows), jnp.float32),
            pltpu.VMEM((2, tile_rows, D), jnp.float32),
            pltpu.VMEM((2, tile_rows, D), jnp.float32),
            pltpu.SemaphoreType.DMA((2,)),
            pltpu.SemaphoreType.DMA((2,)),
        ],
    )(xf, dictionary)

    return (xc.reshape(B, N, D), xr.reshape(B, N, D), aux[0, 0])
